# baseline (speedup 1.0000x reference)
import functools

import jax
import jax.numpy as jnp
import numpy as np
from jax import lax
from jax.experimental import pallas as pl
from jax.experimental.pallas import tpu as pltpu
from jax.experimental.pallas import tpu_sc as plsc

BATCH = 16384
NUM_NUMERIC = 13
N_CAT = 26
VOCAB = 100000
EMB_DIM = 32
OUT_DIM = NUM_NUMERIC + N_CAT * EMB_DIM

_INFO = plsc.get_sparse_core_info()
NC = _INFO.num_cores
NS = _INFO.num_subcores
NW = NC * NS
BW = BATCH // NW
CH = 2
NCHUNK = BW // CH
IPC = CH * N_CAT
IPC_PAD = 64

_mesh = plsc.VectorSubcoreMesh(core_axis_name="c", subcore_axis_name="s")


@functools.partial(
    pl.kernel,
    mesh=_mesh,
    compiler_params=pltpu.CompilerParams(
        use_tc_tiling_on_sc=True, needs_layout_passes=False
    ),
    out_type=jax.ShapeDtypeStruct((BATCH, OUT_DIM), jnp.float32),
    scratch_types=[
        pltpu.VMEM((2, IPC_PAD), jnp.int32),
        pltpu.VMEM((2, IPC_PAD), jnp.int32),
        pltpu.VMEM((2 * IPC, 8, EMB_DIM), jnp.float32),
        pltpu.VMEM((CH, 16), jnp.float32),
        pltpu.VMEM((2 * CH, OUT_DIM), jnp.float32),
        pltpu.SemaphoreType.DMA,
        pltpu.SemaphoreType.DMA,
        pltpu.SemaphoreType.DMA,
    ],
)
def _featurize(numeric_hbm, idx_hbm, tbl_hbm, out_hbm,
               idx_v, subs_v, blk_v, num_v, ob_v, sem, sem2, sem3):
    wid = lax.axis_index("s") * NC + lax.axis_index("c")
    lanes = lax.broadcasted_iota(jnp.int32, (16,), 0)

    def scalar_at(vec, j):
        return lax.reduce_sum_p.bind(
            jnp.where(lanes == j, vec, 0), axes=(0,)
        )

    def load_idx(q, slot):
        pltpu.async_copy(idx_hbm.at[q], idx_v.at[slot], sem2).wait()
        for g in range(IPC_PAD // 16):
            subs_v[slot, pl.ds(g * 16, 16)] = (
                idx_v[slot, pl.ds(g * 16, 16)] & 7
            )

    def fire(slot):
        for g in range((IPC + 15) // 16):
            vec = idx_v[slot, pl.ds(g * 16, 16)]
            for j in range(min(16, IPC - g * 16)):
                flat = scalar_at(vec, j)
                s0 = pl.multiple_of((flat // 8) * 8, 8)
                pltpu.async_copy(
                    tbl_hbm.at[pl.ds(s0, 8), :],
                    blk_v.at[slot * IPC + g * 16 + j], sem,
                )

    def process(q, slot):
        pltpu.make_async_copy(
            tbl_hbm.at[pl.ds(0, IPC * 8), :],
            blk_v.at[pl.ds(slot * IPC, IPC)], sem
        ).wait()
        base = q * CH
        pltpu.async_copy(numeric_hbm.at[pl.ds(base, CH)], num_v, sem2).wait()
        for r in range(CH):
            row = slot * CH + r
            plsc.store_scatter(
                ob_v, [jnp.full((16,), row, jnp.int32), lanes],
                num_v[r, pl.ds(0, 16)]
            )
            for m in range(N_CAT * EMB_DIM // 16):
                f, half = divmod(m, 2)
                k = slot * IPC + r * N_CAT + f
                kc = jnp.full((16,), k, jnp.int32)
                dc = lanes + (16 * half)
                cc = lanes + (NUM_NUMERIC + 16 * m)
                subs16 = plsc.load_gather(
                    subs_v, [jnp.full((16,), slot, jnp.int32),
                             jnp.full((16,), r * N_CAT + f, jnp.int32)]
                )
                val = plsc.load_gather(blk_v, [kc, subs16, dc])
                plsc.store_scatter(
                    ob_v, [jnp.full((16,), row, jnp.int32), cc], val
                )
        pltpu.async_copy(
            ob_v.at[pl.ds(slot * CH, CH)], out_hbm.at[pl.ds(base, CH)], sem3
        )

    def wait_out():
        pltpu.make_async_copy(
            ob_v.at[pl.ds(0, CH)], out_hbm.at[pl.ds(0, CH)], sem3
        ).wait()

    qbase = wid * NCHUNK

    load_idx(qbase, 0)
    fire(0)

    def pair_body(i, _):
        q0 = qbase + 2 * i

        @pl.when(2 * i + 1 < NCHUNK)
        def _():
            load_idx(q0 + 1, 1)
            fire(1)
        process(q0, 0)

        @pl.when(2 * i + 2 < NCHUNK)
        def _():
            load_idx(q0 + 2, 0)
            fire(0)

        @pl.when(2 * i + 1 < NCHUNK)
        def _():
            process(q0 + 1, 1)
        wait_out()
        wait_out()
        return ()

    lax.fori_loop(0, NCHUNK // 2, pair_body, ())


def kernel(numeric, categorical, tables):
    flat = categorical.astype(jnp.int32) + jnp.arange(
        N_CAT, dtype=jnp.int32
    ) * VOCAB
    idx = flat.reshape(NW * NCHUNK, IPC)
    idx = jnp.pad(idx, ((0, 0), (0, IPC_PAD - IPC)))
    tbl = tables.reshape(N_CAT * VOCAB, EMB_DIM)
    num_pad = jnp.pad(numeric, ((0, 0), (0, 16 - NUM_NUMERIC)))
    return _featurize(num_pad, idx, tbl)


# speedup vs baseline: 1.0306x; 1.0306x over previous
import functools

import jax
import jax.numpy as jnp
import numpy as np
from jax import lax
from jax.experimental import pallas as pl
from jax.experimental.pallas import tpu as pltpu
from jax.experimental.pallas import tpu_sc as plsc

BATCH = 16384
NUM_NUMERIC = 13
N_CAT = 26
VOCAB = 100000
EMB_DIM = 32
OUT_DIM = NUM_NUMERIC + N_CAT * EMB_DIM

_INFO = plsc.get_sparse_core_info()
NC = _INFO.num_cores
NS = _INFO.num_subcores
NW = NC * NS
BW = BATCH // NW
CH = 2
NCHUNK = BW // CH
IPC = CH * N_CAT
IPC_PAD = 64

_mesh = plsc.VectorSubcoreMesh(core_axis_name="c", subcore_axis_name="s")


@functools.partial(
    pl.kernel,
    mesh=_mesh,
    compiler_params=pltpu.CompilerParams(
        use_tc_tiling_on_sc=True, needs_layout_passes=False
    ),
    out_type=jax.ShapeDtypeStruct((BATCH, OUT_DIM), jnp.float32),
    scratch_types=[
        pltpu.VMEM((2, IPC_PAD), jnp.int32),
        pltpu.VMEM((2, IPC_PAD), jnp.int32),
        pltpu.VMEM((2 * IPC, 8, EMB_DIM), jnp.float32),
        pltpu.VMEM((CH, 16), jnp.float32),
        pltpu.VMEM((2 * CH, OUT_DIM), jnp.float32),
        pltpu.SemaphoreType.DMA,
        pltpu.SemaphoreType.DMA,
        pltpu.SemaphoreType.DMA,
    ],
)
def _featurize(numeric_hbm, idx_hbm, tbl_hbm, out_hbm,
               idx_v, subs_v, blk_v, num_v, ob_v, sem, sem2, sem3):
    wid = lax.axis_index("s") * NC + lax.axis_index("c")
    lanes = lax.broadcasted_iota(jnp.int32, (16,), 0)

    def scalar_at(vec, j):
        return lax.reduce_sum_p.bind(
            jnp.where(lanes == j, vec, 0), axes=(0,)
        )

    def load_idx(q, slot):
        pltpu.async_copy(idx_hbm.at[q], idx_v.at[slot], sem2).wait()

    def fire(slot):
        for g in range((IPC + 15) // 16):
            vec = idx_v[slot, pl.ds(g * 16, 16)]
            for j in range(min(16, IPC - g * 16)):
                flat = scalar_at(vec, j)
                s0 = pl.multiple_of((flat // 8) * 8, 8)
                pltpu.async_copy(
                    tbl_hbm.at[pl.ds(s0, 8), :],
                    blk_v.at[slot * IPC + g * 16 + j], sem,
                )

    def process(q, slot):
        for k in range(IPC):
            pltpu.make_async_copy(
                tbl_hbm.at[pl.ds(0, 8), :], blk_v.at[slot * IPC + k], sem
            ).wait()
        base = q * CH
        pltpu.async_copy(numeric_hbm.at[pl.ds(base, CH)], num_v, sem2).wait()
        for r in range(CH):
            plsc.store_scatter(
                ob_v, [jnp.full((16,), slot * CH + r, jnp.int32), lanes],
                num_v[r, pl.ds(0, 16)]
            )
        for g in range((IPC + 15) // 16):
            vec = idx_v[slot, pl.ds(g * 16, 16)]
            for j in range(min(16, IPC - g * 16)):
                k = g * 16 + j
                r, f = divmod(k, N_CAT)
                sub = lax.rem(scalar_at(vec, j), 8)
                col = NUM_NUMERIC + f * EMB_DIM
                rvec = jnp.full((16,), slot * CH + r, jnp.int32)
                bk = slot * IPC + k
                plsc.store_scatter(
                    ob_v, [rvec, lanes + col], blk_v[bk, sub, pl.ds(0, 16)]
                )
                plsc.store_scatter(
                    ob_v, [rvec, lanes + (col + 16)],
                    blk_v[bk, sub, pl.ds(16, 16)]
                )
        pltpu.async_copy(
            ob_v.at[pl.ds(slot * CH, CH)], out_hbm.at[pl.ds(base, CH)], sem3
        )

    def wait_out():
        pltpu.make_async_copy(
            ob_v.at[pl.ds(0, CH)], out_hbm.at[pl.ds(0, CH)], sem3
        ).wait()

    qbase = wid * NCHUNK

    load_idx(qbase, 0)
    fire(0)

    def pair_body(i, _):
        q0 = qbase + 2 * i

        @pl.when(2 * i + 1 < NCHUNK)
        def _():
            load_idx(q0 + 1, 1)
            fire(1)
        process(q0, 0)

        @pl.when(2 * i + 2 < NCHUNK)
        def _():
            load_idx(q0 + 2, 0)
            fire(0)

        @pl.when(2 * i + 1 < NCHUNK)
        def _():
            process(q0 + 1, 1)
        wait_out()
        wait_out()
        return ()

    lax.fori_loop(0, NCHUNK // 2, pair_body, ())


def kernel(numeric, categorical, tables):
    flat = categorical.astype(jnp.int32) + jnp.arange(
        N_CAT, dtype=jnp.int32
    ) * VOCAB
    idx = flat.reshape(NW * NCHUNK, IPC)
    idx = jnp.pad(idx, ((0, 0), (0, IPC_PAD - IPC)))
    tbl = tables.reshape(N_CAT * VOCAB, EMB_DIM)
    num_pad = jnp.pad(numeric, ((0, 0), (0, 16 - NUM_NUMERIC)))
    return _featurize(num_pad, idx, tbl)
